# Initial kernel scaffold; baseline (speedup 1.0000x reference)
import jax
import jax.numpy as jnp
from jax.experimental import pallas as pl


def kernel(node_feats, edge_feats, edge_index, edge_indices, W1, b1, W2, b2, conv_bias, Wc1, bc1, Wc2, bc2):
    raise NotImplementedError("write your pallas kernel here")


# speedup vs baseline: 1.0797x; 1.0797x over previous
import functools

import jax
import jax.numpy as jnp
from jax.experimental import pallas as pl


def _mlp_msg_body(ef_ref, xs_ref, W1_ref, b1_ref, W2_ref, b2_ref, R_ref, S_ref,
                  out_ref):
    hid = jnp.maximum(
        jnp.dot(ef_ref[...], W1_ref[...], preferred_element_type=jnp.float32)
        + b1_ref[...], 0.0)
    we = jnp.dot(hid, W2_ref[...], preferred_element_type=jnp.float32) + b2_ref[...]
    xr = jnp.dot(xs_ref[...], R_ref[...], preferred_element_type=jnp.float32)
    out_ref[...] = jnp.dot(xr * we, S_ref[...],
                           preferred_element_type=jnp.float32)


def _cls_body(sh_ref, dh_ref, ef_ref, A_ref, B_ref, C_ref, bc1_ref, Wc2_ref,
              bc2_ref, out_ref):
    z = jnp.maximum(
        jnp.dot(sh_ref[...], A_ref[...], preferred_element_type=jnp.float32)
        + jnp.dot(dh_ref[...], B_ref[...], preferred_element_type=jnp.float32)
        + jnp.dot(ef_ref[...], C_ref[...], preferred_element_type=jnp.float32)
        + bc1_ref[...], 0.0)
    out_ref[...] = jnp.dot(z, Wc2_ref[...],
                           preferred_element_type=jnp.float32) + bc2_ref[...]


def kernel(node_feats, edge_feats, edge_index, edge_indices, W1, b1, W2, b2,
           conv_bias, Wc1, bc1, Wc2, bc2):
    N, IN = node_feats.shape
    E, EF = edge_feats.shape
    H = conv_bias.shape[0]
    K = W1.shape[1]
    HI = W2.shape[1]
    NSUP = edge_indices.shape[0]
    OUT = Wc2.shape[1]

    src = edge_index[0]
    dst = edge_index[1]

    BE = 1024
    E_pad = ((E + BE - 1) // BE) * BE

    R = (jnp.arange(HI)[None, :] // H == jnp.arange(IN)[:, None]).astype(jnp.float32)
    S = (jnp.arange(HI)[:, None] % H == jnp.arange(H)[None, :]).astype(jnp.float32)

    xs = node_feats[src]
    xs_p = jnp.pad(xs, ((0, E_pad - E), (0, 0)))
    ef_p = jnp.pad(edge_feats, ((0, E_pad - E), (0, 0)))

    grid = (E_pad // BE,)
    msg_p = pl.pallas_call(
        _mlp_msg_body,
        grid=grid,
        in_specs=[
            pl.BlockSpec((BE, EF), lambda i: (i, 0)),
            pl.BlockSpec((BE, IN), lambda i: (i, 0)),
            pl.BlockSpec((EF, K), lambda i: (0, 0)),
            pl.BlockSpec((1, K), lambda i: (0, 0)),
            pl.BlockSpec((K, HI), lambda i: (0, 0)),
            pl.BlockSpec((1, HI), lambda i: (0, 0)),
            pl.BlockSpec((IN, HI), lambda i: (0, 0)),
            pl.BlockSpec((HI, H), lambda i: (0, 0)),
        ],
        out_specs=pl.BlockSpec((BE, H), lambda i: (i, 0)),
        out_shape=jax.ShapeDtypeStruct((E_pad, H), jnp.float32),
    )(ef_p, xs_p, W1, b1.reshape(1, K), W2, b2.reshape(1, HI), R, S)
    msg = msg_p[:E]

    deg = jax.ops.segment_sum(jnp.ones((E,), jnp.float32), dst, num_segments=N)
    agg = jax.ops.segment_sum(msg, dst, num_segments=N)
    h = jnp.maximum(agg / jnp.maximum(deg, 1.0)[:, None] + conv_bias, 0.0)

    NS_pad = ((NSUP + 1023) // 1024) * 1024
    eidx = jnp.pad(edge_indices, (0, NS_pad - NSUP))
    s_idx = src[eidx]
    d_idx = dst[eidx]
    sh = h[s_idx]
    dh = h[d_idx]
    efx = edge_feats[eidx]

    logits_p = pl.pallas_call(
        _cls_body,
        in_specs=[
            pl.BlockSpec((NS_pad, H), lambda: (0, 0)),
            pl.BlockSpec((NS_pad, H), lambda: (0, 0)),
            pl.BlockSpec((NS_pad, EF), lambda: (0, 0)),
            pl.BlockSpec((H, H), lambda: (0, 0)),
            pl.BlockSpec((H, H), lambda: (0, 0)),
            pl.BlockSpec((EF, H), lambda: (0, 0)),
            pl.BlockSpec((1, H), lambda: (0, 0)),
            pl.BlockSpec((H, OUT), lambda: (0, 0)),
            pl.BlockSpec((1, OUT), lambda: (0, 0)),
        ],
        out_specs=pl.BlockSpec((NS_pad, OUT), lambda: (0, 0)),
        out_shape=jax.ShapeDtypeStruct((NS_pad, OUT), jnp.float32),
    )(sh, dh, efx, Wc1[:H], Wc1[H:2 * H], Wc1[2 * H:], bc1.reshape(1, H),
      Wc2, bc2.reshape(1, OUT))
    return logits_p[:NSUP]


# speedup vs baseline: 1.8745x; 1.7360x over previous
import functools

import jax
import jax.numpy as jnp
from jax import lax
from jax.experimental import pallas as pl
from jax.experimental.pallas import tpu as pltpu
from jax.experimental.pallas import tpu_sc as plsc

_NC, _NS = 2, 16
_NW = _NC * _NS
_GB = 5


def _mlp_msg_body(E, BE, ef_ref, xs_ref, W1_ref, b1_ref, W2_ref, b2_ref,
                  R_ref, S_ref, F_ref, out_ref):
    hid = jnp.maximum(
        jnp.dot(ef_ref[...], W1_ref[...], preferred_element_type=jnp.float32)
        + b1_ref[...], 0.0)
    we = jnp.dot(hid, W2_ref[...], preferred_element_type=jnp.float32) + b2_ref[...]
    xr = jnp.dot(xs_ref[...], R_ref[...], preferred_element_type=jnp.float32)
    msg = jnp.dot(xr * we, S_ref[...], preferred_element_type=jnp.float32)
    row = pl.program_id(0) * BE + lax.broadcasted_iota(jnp.int32, (BE, 1), 0)
    valid = (row < E).astype(jnp.float32)
    out_ref[...] = msg + valid * F_ref[...]


def _scatter_body(n_chunks, zrows_pt, msgv4, dst3, zrows, out, idx_v, val_v,
                  acc_sh):
    cid = lax.axis_index("c")
    sid = lax.axis_index("s")
    wid = sid * _NC + cid
    pltpu.sync_copy(zrows, acc_sh.at[pl.ds(sid * zrows_pt, zrows_pt)])
    plsc.subcore_barrier()
    pltpu.sync_copy(dst3.at[wid], idx_v)

    def body(g, carry):
        pltpu.sync_copy(msgv4.at[wid, pl.ds(g * _GB, _GB)], val_v)
        for b in range(_GB):
            pltpu.sync_copy(val_v.at[b], acc_sh.at[idx_v.at[g * _GB + b]],
                            add=True)
        return carry

    lax.fori_loop(0, n_chunks // _GB, body, 0)
    plsc.subcore_barrier()
    pltpu.sync_copy(acc_sh.at[pl.ds(sid * zrows_pt, zrows_pt)],
                    out.at[cid, pl.ds(sid * zrows_pt, zrows_pt)])


def _finalize_body(parts_ref, bias_ref, out_ref):
    a = parts_ref[0] + parts_ref[1]
    m = a[:, :16]
    deg = a[:, 16:17]
    out_ref[...] = jnp.maximum(m / jnp.maximum(deg, 1.0) + bias_ref[...], 0.0)


def _cls_body(sh_ref, dh_ref, ef_ref, A_ref, B_ref, C_ref, bc1_ref, Wc2_ref,
              bc2_ref, out_ref):
    z = jnp.maximum(
        jnp.dot(sh_ref[...], A_ref[...], preferred_element_type=jnp.float32)
        + jnp.dot(dh_ref[...], B_ref[...], preferred_element_type=jnp.float32)
        + jnp.dot(ef_ref[...], C_ref[...], preferred_element_type=jnp.float32)
        + bc1_ref[...], 0.0)
    out_ref[...] = jnp.dot(z, Wc2_ref[...],
                           preferred_element_type=jnp.float32) + bc2_ref[...]


def kernel(node_feats, edge_feats, edge_index, edge_indices, W1, b1, W2, b2,
           conv_bias, Wc1, bc1, Wc2, bc2):
    N, IN = node_feats.shape
    E, EF = edge_feats.shape
    H = conv_bias.shape[0]
    K = W1.shape[1]
    HI = W2.shape[1]
    NSUP = edge_indices.shape[0]
    OUT = Wc2.shape[1]
    HW = 32

    src = edge_index[0]
    dst = edge_index[1]

    BE = 1024
    E_pad = ((E + _NW * 128 - 1) // (_NW * 128)) * (_NW * 128)
    assert E_pad % BE == 0
    rows_per_tile = E_pad // _NW
    n_chunks = rows_per_tile // 128

    R = (jnp.arange(HI)[None, :] // H == jnp.arange(IN)[:, None]).astype(jnp.float32)
    S = (jnp.arange(HI)[:, None] % H == jnp.arange(H)[None, :]).astype(jnp.float32)
    S = jnp.concatenate([S, jnp.zeros((HI, HW - H), jnp.float32)], axis=1)
    F = (jnp.arange(HW)[None, :] == H).astype(jnp.float32)

    xs = node_feats[src]
    xs_p = jnp.pad(xs, ((0, E_pad - E), (0, 0)))
    ef_p = jnp.pad(edge_feats, ((0, E_pad - E), (0, 0)))

    grid = (E_pad // BE,)
    msgv = pl.pallas_call(
        functools.partial(_mlp_msg_body, E, BE),
        grid=grid,
        in_specs=[
            pl.BlockSpec((BE, EF), lambda i: (i, 0)),
            pl.BlockSpec((BE, IN), lambda i: (i, 0)),
            pl.BlockSpec((EF, K), lambda i: (0, 0)),
            pl.BlockSpec((1, K), lambda i: (0, 0)),
            pl.BlockSpec((K, HI), lambda i: (0, 0)),
            pl.BlockSpec((1, HI), lambda i: (0, 0)),
            pl.BlockSpec((IN, HI), lambda i: (0, 0)),
            pl.BlockSpec((HI, HW), lambda i: (0, 0)),
            pl.BlockSpec((1, HW), lambda i: (0, 0)),
        ],
        out_specs=pl.BlockSpec((BE, HW), lambda i: (i, 0)),
        out_shape=jax.ShapeDtypeStruct((E_pad, HW), jnp.float32),
    )(ef_p, xs_p, W1, b1.reshape(1, K), W2, b2.reshape(1, HI), R, S, F)

    N_pad = ((N + _NS * 8 - 1) // (_NS * 8)) * (_NS * 8)
    zrows_pt = N_pad // _NS
    dst3 = jnp.pad(dst, (0, E_pad - E)).reshape(_NW, n_chunks, 128)
    msgv4 = msgv.reshape(_NW, n_chunks, 128, HW)
    zrows = jnp.zeros((zrows_pt, HW), jnp.float32)

    scatter = functools.partial(
        pl.kernel,
        mesh=plsc.VectorSubcoreMesh(core_axis_name="c", subcore_axis_name="s"),
        compiler_params=pltpu.CompilerParams(use_tc_tiling_on_sc=False),
        out_type=jax.ShapeDtypeStruct((_NC, N_pad, HW), jnp.float32),
        scratch_types=[
            pltpu.VMEM((n_chunks, 128), jnp.int32),
            pltpu.VMEM((_GB, 128, HW), jnp.float32),
            pltpu.VMEM_SHARED((N_pad, HW), jnp.float32),
        ],
    )(functools.partial(_scatter_body, n_chunks, zrows_pt))
    parts = scatter(msgv4, dst3, zrows)

    h_pad = pl.pallas_call(
        _finalize_body,
        in_specs=[
            pl.BlockSpec((_NC, N_pad, HW), lambda: (0, 0, 0)),
            pl.BlockSpec((1, H), lambda: (0, 0)),
        ],
        out_specs=pl.BlockSpec((N_pad, H), lambda: (0, 0)),
        out_shape=jax.ShapeDtypeStruct((N_pad, H), jnp.float32),
    )(parts, conv_bias.reshape(1, H))
    h = h_pad[:N]

    NS_pad = ((NSUP + 1023) // 1024) * 1024
    eidx = jnp.pad(edge_indices, (0, NS_pad - NSUP))
    s_idx = src[eidx]
    d_idx = dst[eidx]
    sh = h[s_idx]
    dh = h[d_idx]
    efx = edge_feats[eidx]

    logits_p = pl.pallas_call(
        _cls_body,
        in_specs=[
            pl.BlockSpec((NS_pad, H), lambda: (0, 0)),
            pl.BlockSpec((NS_pad, H), lambda: (0, 0)),
            pl.BlockSpec((NS_pad, EF), lambda: (0, 0)),
            pl.BlockSpec((H, H), lambda: (0, 0)),
            pl.BlockSpec((H, H), lambda: (0, 0)),
            pl.BlockSpec((EF, H), lambda: (0, 0)),
            pl.BlockSpec((1, H), lambda: (0, 0)),
            pl.BlockSpec((H, OUT), lambda: (0, 0)),
            pl.BlockSpec((1, OUT), lambda: (0, 0)),
        ],
        out_specs=pl.BlockSpec((NS_pad, OUT), lambda: (0, 0)),
        out_shape=jax.ShapeDtypeStruct((NS_pad, OUT), jnp.float32),
    )(sh, dh, efx, Wc1[:H], Wc1[H:2 * H], Wc1[2 * H:], bc1.reshape(1, H),
      Wc2, bc2.reshape(1, OUT))
    return logits_p[:NSUP]
